# Initial kernel scaffold; baseline (speedup 1.0000x reference)
import jax
import jax.numpy as jnp
from jax.experimental import pallas as pl


def kernel(x, W):
    raise NotImplementedError("write your pallas kernel here")


# speedup vs baseline: 2.2977x; 2.2977x over previous
import functools

import jax
import jax.numpy as jnp
from jax import lax
from jax.experimental import pallas as pl
from jax.experimental.pallas import tpu as pltpu
from jax.experimental.pallas import tpu_sc as plsc

NC = 2
NS = 16
NW = NC * NS

CHUNK = 512
SUB = 128


def _make_lookup(total: int, D: int):
  per_tile = total // NW
  n_chunks = per_tile // CHUNK
  mesh = plsc.VectorSubcoreMesh(core_axis_name="c", subcore_axis_name="s")

  @functools.partial(
      pl.kernel,
      out_type=jax.ShapeDtypeStruct((total, D), jnp.float32),
      mesh=mesh,
      scratch_types=[
          pltpu.VMEM((CHUNK,), jnp.int32),
          pltpu.VMEM((CHUNK, D), jnp.float32),
          pltpu.SemaphoreType.DMA,
      ],
      compiler_params=pltpu.CompilerParams(use_tc_tiling_on_sc=False),
  )
  def lookup(x_hbm, w_hbm, out_hbm, idx_v, rows_v, sem):
    wid = lax.axis_index("s") * NC + lax.axis_index("c")
    base = wid * per_tile

    def step(g, carry):
      off = base + g * CHUNK
      pltpu.sync_copy(x_hbm.at[pl.ds(off, CHUNK)], idx_v)
      copies = []
      for j in range(CHUNK // SUB):
        copies.append(
            pltpu.async_copy(
                w_hbm.at[idx_v.at[pl.ds(j * SUB, SUB)]],
                rows_v.at[pl.ds(j * SUB, SUB)],
                sem,
            ))
      for c in copies:
        c.wait()
      pltpu.sync_copy(rows_v, out_hbm.at[pl.ds(off, CHUNK)])
      return carry

    lax.fori_loop(0, n_chunks, step, 0)

  return lookup


def kernel(x, W):
  B, L = x.shape
  N, D = W.shape
  total = B * L
  x_flat = x.reshape(total).astype(jnp.int32)
  out = _make_lookup(total, D)(x_flat, W)
  return out.reshape(B, L, D)


# speedup vs baseline: 2.5350x; 1.1033x over previous
import functools

import jax
import jax.numpy as jnp
from jax import lax
from jax.experimental import pallas as pl
from jax.experimental.pallas import tpu as pltpu
from jax.experimental.pallas import tpu_sc as plsc

NC = 2
NS = 16
NW = NC * NS

CHUNK = 2048
SUB = 128


def _make_lookup(total: int, D: int):
  per_tile = total // NW
  n_chunks = per_tile // CHUNK
  assert n_chunks % 2 == 0 and n_chunks >= 4
  n_pairs = n_chunks // 2
  mesh = plsc.VectorSubcoreMesh(core_axis_name="c", subcore_axis_name="s")

  @functools.partial(
      pl.kernel,
      out_type=jax.ShapeDtypeStruct((total, D), jnp.float32),
      mesh=mesh,
      scratch_types=[
          pltpu.VMEM((CHUNK,), jnp.int32),
          pltpu.VMEM((CHUNK,), jnp.int32),
          pltpu.VMEM((CHUNK, D), jnp.float32),
          pltpu.VMEM((CHUNK, D), jnp.float32),
          pltpu.SemaphoreType.DMA,
          pltpu.SemaphoreType.DMA,
          pltpu.SemaphoreType.DMA,
          pltpu.SemaphoreType.DMA,
          pltpu.SemaphoreType.DMA,
          pltpu.SemaphoreType.DMA,
      ],
      compiler_params=pltpu.CompilerParams(use_tc_tiling_on_sc=False),
  )
  def lookup(x_hbm, w_hbm, out_hbm, idx0, idx1, rows0, rows1,
             si0, si1, sg0, sg1, so0, so1):
    wid = lax.axis_index("s") * NC + lax.axis_index("c")
    base = wid * per_tile
    idx_b = (idx0, idx1)
    rows_b = (rows0, rows1)
    si = (si0, si1)
    sg = (sg0, sg1)
    so = (so0, so1)

    def idx_copy(g, slot):
      off = base + g * CHUNK
      pltpu.async_copy(x_hbm.at[pl.ds(off, CHUNK)], idx_b[slot], si[slot])

    idx_copy(0, 0)
    idx_copy(1, 1)

    def pair(p, carry):
      for slot in (0, 1):
        g = 2 * p + slot
        off = base + g * CHUNK

        @pl.when(p > 0)
        def _():
          pltpu.make_async_copy(
              rows_b[slot], out_hbm.at[pl.ds(base, CHUNK)], so[slot]).wait()

        pltpu.make_async_copy(
            x_hbm.at[pl.ds(off, CHUNK)], idx_b[slot], si[slot]).wait()

        gathers = [
            pltpu.async_copy(
                w_hbm.at[idx_b[slot].at[pl.ds(j * SUB, SUB)]],
                rows_b[slot].at[pl.ds(j * SUB, SUB)],
                sg[slot],
            )
            for j in range(CHUNK // SUB)
        ]

        for cp in gathers:
          cp.wait()

        @pl.when(g + 2 < n_chunks)
        def _():
          idx_copy(g + 2, slot)

        pltpu.async_copy(rows_b[slot], out_hbm.at[pl.ds(off, CHUNK)], so[slot])
      return carry

    lax.fori_loop(0, n_pairs, pair, 0)

    for slot in (0, 1):
      pltpu.make_async_copy(
          rows_b[slot], out_hbm.at[pl.ds(base, CHUNK)], so[slot]).wait()

  return lookup


def kernel(x, W):
  B, L = x.shape
  N, D = W.shape
  total = B * L
  x_flat = x.reshape(total).astype(jnp.int32)
  out = _make_lookup(total, D)(x_flat, W)
  return out.reshape(B, L, D)
